# baseline (speedup 1.0000x reference)
import functools

import jax
import jax.numpy as jnp
from jax import lax
from jax.experimental import pallas as pl
from jax.experimental.pallas import tpu as pltpu
from jax.experimental.pallas import tpu_sc as plsc

DIM = 1024
NUM_ROWS = 5
LANES = 16
NC, NS = 2, 16
NW = NC * NS
REPB = 48
POSCAP = 1088
QMAX_S = 8
QMAX = 16
BIG = 1 << 30
BIGF = 3.4e38


def _sc_embed(ids_flat, tbl_flat, scale, n):
    n_per_w = n // NW
    nvec = n_per_w // LANES
    mesh = plsc.VectorSubcoreMesh(
        core_axis_name="c", subcore_axis_name="s", num_cores=NC, num_subcores=NS
    )

    @functools.partial(
        pl.kernel,
        out_type=jax.ShapeDtypeStruct((n, DIM), jnp.float32),
        mesh=mesh,
        compiler_params=pltpu.CompilerParams(needs_layout_passes=False),
        scratch_types=[
            pltpu.VMEM((n_per_w,), jnp.int32),
            pltpu.VMEM((LANES,), jnp.float32),
            pltpu.VMEM((NUM_ROWS * DIM,), jnp.float32),
            pltpu.VMEM((REPB, DIM), jnp.float32),
            pltpu.VMEM((REPB, DIM), jnp.float32),
            pltpu.VMEM((NUM_ROWS * POSCAP,), jnp.int32),
            pltpu.SemaphoreType.DMA,
            pltpu.SemaphoreType.DMA,
            pltpu.SemaphoreType.DMA,
            pltpu.SemaphoreType.DMA,
        ],
    )
    def k(ids_hbm, tbl_hbm, scl_hbm, out_hbm, idx_v, scl_v, tbl_v,
          blk_a, blk_b, pos_v, sem_sa, sem_sb, sem_ta, sem_tb):
        wid = lax.axis_index("s") * NC + lax.axis_index("c")
        base = wid * n_per_w
        pltpu.sync_copy(ids_hbm.at[pl.ds(base, n_per_w)], idx_v)
        pltpu.sync_copy(scl_hbm, scl_v.at[pl.ds(0, 1)])
        pltpu.sync_copy(tbl_hbm, tbl_v)
        iota16 = lax.iota(jnp.int32, LANES)

        raw = scl_v[...]
        sval = jnp.min(jnp.where(iota16 == 0, raw, jnp.float32(BIGF)))
        sv = jnp.broadcast_to(sval, (LANES,))

        def scale_slice(j, _):
            tbl_v[pl.ds(j * LANES, LANES)] = tbl_v[pl.ds(j * LANES, LANES)] * sv
            return 0
        lax.fori_loop(0, NUM_ROWS * DIM // LANES, scale_slice, 0)

        bufs = ((blk_a, sem_sa, sem_ta), (blk_b, sem_sb, sem_tb))

        def wait_main(blk, sem):
            pltpu.make_async_copy(
                blk.at[pl.ds(0, REPB)], out_hbm.at[pl.ds(0, REPB)], sem
            ).wait()

        def wait_tail(blk, sem):
            pltpu.make_async_copy(
                blk.at[pl.ds(0, LANES)], out_hbm.at[pl.ds(0, LANES)], sem
            ).wait()

        def modality(m, state, blk, sem_s, sem_t):
            start = m * POSCAP
            is_, ws_, it_, wt_ = state

            def drain_s(i, _):
                wait_main(blk, sem_s)
                return 0
            lax.fori_loop(0, is_ - ws_, drain_s, 0)

            def drain_t(i, _):
                wait_tail(blk, sem_t)
                return 0
            lax.fori_loop(0, it_ - wt_, drain_t, 0)

            def rep_body(r, _):
                def cp_r(j, _):
                    blk[r, pl.ds(j * LANES, LANES)] = tbl_v[
                        pl.ds(m * DIM + j * LANES, LANES)
                    ]
                    return 0
                lax.fori_loop(0, DIM // LANES, cp_r, 0, unroll=8)
                return 0
            lax.fori_loop(0, REPB, rep_body, 0)

            def comp(v, cnt):
                ids16 = idx_v[pl.ds(v * LANES, LANES)]
                mask = ids16 == m
                posv = (base + v * LANES) + iota16
                plsc.store_compressed(pos_v.at[pl.ds(start + cnt, LANES)],
                                      posv, mask=mask)
                return cnt + jnp.sum(mask.astype(jnp.int32))

            cnt = lax.fori_loop(0, nvec, comp, jnp.int32(0))

            fl = (cnt >> 4) << 4
            head = pos_v[pl.ds(start, LANES)]
            valid_head = jnp.where(iota16 < jnp.minimum(cnt, LANES), head, BIG)
            pad = jnp.broadcast_to(jnp.min(valid_head), (LANES,))
            tail = pos_v[pl.ds(start + fl, LANES)]
            pos_v[pl.ds(start + fl, LANES)] = jnp.where(
                iota16 < (cnt & 15), tail, pad
            )

            t_main = cnt // REPB
            t_tail = ((cnt - t_main * REPB) + 15) >> 4

            def scat_main(t, carry):
                is_, ws_ = carry
                idxref = pos_v.at[pl.ds(start + t * REPB, REPB)]
                pltpu.async_copy(
                    blk.at[pl.ds(0, REPB)], out_hbm.at[idxref], sem_s
                )
                is_ = is_ + 1

                def throttle(w):
                    wait_main(blk, sem_s)
                    return w + 1

                ws_ = lax.cond(is_ - ws_ > QMAX_S, throttle, lambda w: w, ws_)
                return is_, ws_

            is_, ws_ = lax.fori_loop(0, t_main, scat_main,
                                     (jnp.int32(0), jnp.int32(0)))
            tail0 = start + t_main * REPB

            def scat_tail(t, carry):
                it_, wt_ = carry
                idxvec = pos_v[pl.ds(tail0 + t * LANES, LANES)]
                pltpu.async_copy(
                    blk.at[pl.ds(0, LANES)], out_hbm.at[idxvec], sem_t
                )
                it_ = it_ + 1

                def throttle(w):
                    wait_tail(blk, sem_t)
                    return w + 1

                wt_ = lax.cond(it_ - wt_ > QMAX, throttle, lambda w: w, wt_)
                return it_, wt_

            it_, wt_ = lax.fori_loop(0, t_tail, scat_tail,
                                     (jnp.int32(0), jnp.int32(0)))
            return (is_, ws_, it_, wt_)

        zero4 = (jnp.int32(0), jnp.int32(0), jnp.int32(0), jnp.int32(0))
        states = [zero4, zero4]
        for m in range(NUM_ROWS):
            blk, sem_s, sem_t = bufs[m % 2]
            states[m % 2] = modality(m, states[m % 2], blk, sem_s, sem_t)

        for p, (blk, sem_s, sem_t) in enumerate(bufs):
            is_, ws_, it_, wt_ = states[p]

            def drain_s(i, _, blk=blk, sem_s=sem_s):
                wait_main(blk, sem_s)
                return 0
            lax.fori_loop(0, is_ - ws_, drain_s, 0)

            def drain_t(i, _, blk=blk, sem_t=sem_t):
                wait_tail(blk, sem_t)
                return 0
            lax.fori_loop(0, it_ - wt_, drain_t, 0)

    return k(ids_flat, tbl_flat, scale)


def kernel(modality_ids, embed, scale):
    b, s = modality_ids.shape
    n = b * s
    ids_flat = modality_ids.reshape(n).astype(jnp.int32)
    tbl_flat = embed.astype(jnp.float32).reshape(NUM_ROWS * DIM)
    out = _sc_embed(ids_flat, tbl_flat, scale.astype(jnp.float32), n)
    return out.reshape(b, s, DIM)


# speedup vs baseline: 1.5346x; 1.5346x over previous
import functools

import jax
import jax.numpy as jnp
from jax import lax
from jax.experimental import pallas as pl
from jax.experimental.pallas import tpu as pltpu
from jax.experimental.pallas import tpu_sc as plsc

DIM = 1024
NUM_ROWS = 5
LANES = 16
NC, NS = 2, 16
NW = NC * NS
REP = 16
POSCAP = 1088
QMAX = 24
BIG = 1 << 30
BIGF = 3.4e38


def _sc_embed(ids_flat, tbl_flat, scale, n):
    n_per_w = n // NW
    nvec = n_per_w // LANES
    mesh = plsc.VectorSubcoreMesh(
        core_axis_name="c", subcore_axis_name="s", num_cores=NC, num_subcores=NS
    )

    @functools.partial(
        pl.kernel,
        out_type=jax.ShapeDtypeStruct((n, DIM), jnp.float32),
        mesh=mesh,
        compiler_params=pltpu.CompilerParams(needs_layout_passes=False),
        scratch_types=[
            pltpu.VMEM((n_per_w,), jnp.int32),
            pltpu.VMEM((LANES,), jnp.float32),
            pltpu.VMEM((NUM_ROWS * DIM,), jnp.float32),
            pltpu.VMEM((NUM_ROWS * REP, DIM), jnp.float32),
            pltpu.VMEM((NUM_ROWS * POSCAP,), jnp.int32),
            pltpu.SemaphoreType.DMA,
        ],
    )
    def k(ids_hbm, tbl_hbm, scl_hbm, out_hbm, idx_v, scl_v, tbl_v,
          blk_v, pos_v, ssem):
        wid = lax.axis_index("s") * NC + lax.axis_index("c")
        base = wid * n_per_w
        pltpu.sync_copy(ids_hbm.at[pl.ds(base, n_per_w)], idx_v)
        pltpu.sync_copy(scl_hbm, scl_v.at[pl.ds(0, 1)])
        pltpu.sync_copy(tbl_hbm, tbl_v)
        iota16 = lax.iota(jnp.int32, LANES)

        raw = scl_v[...]
        sval = jnp.min(jnp.where(iota16 == 0, raw, jnp.float32(BIGF)))
        sv = jnp.broadcast_to(sval, (LANES,))

        def scale_slice(j, _):
            tbl_v[pl.ds(j * LANES, LANES)] = tbl_v[pl.ds(j * LANES, LANES)] * sv
            return 0
        lax.fori_loop(0, NUM_ROWS * DIM // LANES, scale_slice, 0)

        def wait_one():
            pltpu.make_async_copy(
                blk_v.at[pl.ds(0, REP)], out_hbm.at[pl.ds(0, REP)], ssem
            ).wait()

        def modality(m, state):
            start = m * POSCAP

            def rep_body(r, _):
                def cp_r(j, _):
                    blk_v[m * REP + r, pl.ds(j * LANES, LANES)] = tbl_v[
                        pl.ds(m * DIM + j * LANES, LANES)
                    ]
                    return 0
                lax.fori_loop(0, DIM // LANES, cp_r, 0, unroll=8)
                return 0
            lax.fori_loop(0, REP, rep_body, 0)

            def comp(v, cnt):
                ids16 = idx_v[pl.ds(v * LANES, LANES)]
                mask = ids16 == m
                posv = (base + v * LANES) + iota16
                plsc.store_compressed(pos_v.at[pl.ds(start + cnt, LANES)],
                                      posv, mask=mask)
                return cnt + jnp.sum(mask.astype(jnp.int32))

            cnt = lax.fori_loop(0, nvec, comp, jnp.int32(0))

            fl = (cnt >> 4) << 4
            head = pos_v[pl.ds(start, LANES)]
            valid_head = jnp.where(iota16 < jnp.minimum(cnt, LANES), head, BIG)
            pad = jnp.broadcast_to(jnp.min(valid_head), (LANES,))
            tail = pos_v[pl.ds(start + fl, LANES)]
            pos_v[pl.ds(start + fl, LANES)] = jnp.where(
                iota16 < (cnt & 15), tail, pad
            )

            t_m = (cnt + 15) >> 4

            def scat(t, carry):
                issued, waited = carry
                idxvec = pos_v[pl.ds(start + t * LANES, LANES)]
                pltpu.async_copy(
                    blk_v.at[pl.ds(m * REP, REP)], out_hbm.at[idxvec], ssem
                )
                issued = issued + 1

                def throttle(w):
                    wait_one()
                    return w + 1

                waited = lax.cond(issued - waited > QMAX, throttle,
                                  lambda w: w, waited)
                return issued, waited

            return lax.fori_loop(0, t_m, scat, state)

        state = lax.fori_loop(
            0, NUM_ROWS, modality, (jnp.int32(0), jnp.int32(0))
        )
        issued, waited = state

        def drain(i, _):
            wait_one()
            return 0
        lax.fori_loop(0, issued - waited, drain, 0)

    return k(ids_flat, tbl_flat, scale)


def kernel(modality_ids, embed, scale):
    b, s = modality_ids.shape
    n = b * s
    ids_flat = modality_ids.reshape(n).astype(jnp.int32)
    tbl_flat = embed.astype(jnp.float32).reshape(NUM_ROWS * DIM)
    out = _sc_embed(ids_flat, tbl_flat, scale.astype(jnp.float32), n)
    return out.reshape(b, s, DIM)
